# baseline (speedup 1.0000x reference)
import functools

import jax
import jax.numpy as jnp
from jax import lax
from jax.experimental import pallas as pl
from jax.experimental.pallas import tpu as pltpu
from jax.experimental.pallas import tpu_sc as plsc


def _pos_encoding(seq_len, emb_size):
    seq = jnp.arange(1, seq_len + 1, dtype=jnp.float32).reshape(-1, 1)
    pos = jnp.arange(1, emb_size + 1, dtype=jnp.float32)
    pe = seq / jnp.power(10000.0, 2.0 * pos / emb_size)
    pe = pe.at[:, ::2].set(jnp.sin(pe[:, ::2]))
    pe = pe.at[:, 1::2].set(jnp.cos(pe[:, 1::2]))
    return pe


def _sc_gather(idx, table):
    info = plsc.get_sparse_core_info()
    nw = info.num_cores * info.num_subcores
    b = idx.shape[0]
    d = table.shape[1]
    b_per_w = b // nw
    ch = 128
    n_ch = b_per_w // ch

    mesh = plsc.VectorSubcoreMesh(core_axis_name="c", subcore_axis_name="s")

    @functools.partial(
        pl.kernel,
        mesh=mesh,
        out_type=jax.ShapeDtypeStruct((b, d), jnp.float32),
        scratch_types=[
            pltpu.VMEM((b_per_w,), jnp.int32),
            pltpu.VMEM((b_per_w, d), jnp.float32),
            pltpu.SemaphoreType.DMA,
        ],
    )
    def gather_kernel(idx_hbm, table_hbm, out_hbm, idx_v, rows_v, sem):
        wid = lax.axis_index("s") * info.num_cores + lax.axis_index("c")
        base = wid * b_per_w
        pltpu.sync_copy(idx_hbm.at[pl.ds(base, b_per_w)], idx_v)
        copies = [
            pltpu.async_copy(
                table_hbm.at[idx_v.at[pl.ds(j * ch, ch)]],
                rows_v.at[pl.ds(j * ch, ch)],
                sem,
            )
            for j in range(n_ch)
        ]
        for c in copies:
            c.wait()
        pltpu.sync_copy(rows_v, out_hbm.at[pl.ds(base, b_per_w)])

    return gather_kernel(idx, table)


def _tc_project(emb, pe, wq, bq):
    l, e = emb.shape
    o = wq.shape[0]
    br = 1024

    def body(emb_ref, pe_ref, wq_ref, bq_ref, out_ref):
        x = emb_ref[...] + pe_ref[...]
        out_ref[...] = (
            lax.dot_general(
                x, wq_ref[...], (((1,), (1,)), ((), ())),
                preferred_element_type=jnp.float32,
            )
            + bq_ref[...]
        )

    return pl.pallas_call(
        body,
        grid=(l // br,),
        in_specs=[
            pl.BlockSpec((br, e), lambda i: (i, 0)),
            pl.BlockSpec((br, e), lambda i: (i, 0)),
            pl.BlockSpec((o, e), lambda i: (0, 0)),
            pl.BlockSpec((1, o), lambda i: (0, 0)),
        ],
        out_specs=pl.BlockSpec((br, o), lambda i: (i, 0)),
        out_shape=jax.ShapeDtypeStruct((l, o), jnp.float32),
    )(emb, pe, wq, bq.reshape(1, o))


def kernel(inputs, table, Wq, bq):
    emb = _sc_gather(inputs, table)
    pe = _pos_encoding(inputs.shape[0], table.shape[1])
    return _tc_project(emb, pe, Wq, bq)


# speedup vs baseline: 2.9996x; 2.9996x over previous
import functools

import numpy as np

import jax
import jax.numpy as jnp
from jax import lax
from jax.experimental import pallas as pl
from jax.experimental.pallas import tpu as pltpu
from jax.experimental.pallas import tpu_sc as plsc


def _pos_encoding(seq_len, emb_size):
    seq = np.arange(1, seq_len + 1, dtype=np.float64).reshape(-1, 1)
    pos = np.arange(1, emb_size + 1, dtype=np.float64)
    pe = seq / np.power(10000.0, 2.0 * pos / emb_size)
    pe[:, ::2] = np.sin(pe[:, ::2])
    pe[:, 1::2] = np.cos(pe[:, 1::2])
    return pe.astype(np.float32)


_PE = _pos_encoding(16384, 128)


def _sc_gather(idx, table):
    info = plsc.get_sparse_core_info()
    nw = info.num_cores * info.num_subcores
    b = idx.shape[0]
    d = table.shape[1]
    b_per_w = b // nw
    ch = 128
    n_ch = b_per_w // ch

    mesh = plsc.VectorSubcoreMesh(core_axis_name="c", subcore_axis_name="s")

    @functools.partial(
        pl.kernel,
        mesh=mesh,
        out_type=jax.ShapeDtypeStruct((b, d), jnp.float32),
        scratch_types=[
            pltpu.VMEM((b_per_w,), jnp.int32),
            pltpu.VMEM((b_per_w, d), jnp.float32),
            pltpu.SemaphoreType.DMA,
        ],
    )
    def gather_kernel(idx_hbm, table_hbm, out_hbm, idx_v, rows_v, sem):
        wid = lax.axis_index("s") * info.num_cores + lax.axis_index("c")
        base = wid * b_per_w
        pltpu.sync_copy(idx_hbm.at[pl.ds(base, b_per_w)], idx_v)
        copies = [
            pltpu.async_copy(
                table_hbm.at[idx_v.at[pl.ds(j * ch, ch)]],
                rows_v.at[pl.ds(j * ch, ch)],
                sem,
            )
            for j in range(n_ch)
        ]
        for c in copies:
            c.wait()
        pltpu.sync_copy(rows_v, out_hbm.at[pl.ds(base, b_per_w)])

    return gather_kernel(idx, table)


def _tc_project(emb, pe, wq, bq):
    l, e = emb.shape
    o = wq.shape[0]
    br = 1024

    def body(emb_ref, pe_ref, wq_ref, bq_ref, out_ref):
        x = emb_ref[...] + pe_ref[...]
        out_ref[...] = (
            lax.dot_general(
                x, wq_ref[...], (((1,), (1,)), ((), ())),
                preferred_element_type=jnp.float32,
            )
            + bq_ref[...]
        )

    return pl.pallas_call(
        body,
        grid=(l // br,),
        in_specs=[
            pl.BlockSpec((br, e), lambda i: (i, 0)),
            pl.BlockSpec((br, e), lambda i: (i, 0)),
            pl.BlockSpec((o, e), lambda i: (0, 0)),
            pl.BlockSpec((1, o), lambda i: (0, 0)),
        ],
        out_specs=pl.BlockSpec((br, o), lambda i: (i, 0)),
        out_shape=jax.ShapeDtypeStruct((l, o), jnp.float32),
    )(emb, pe, wq, bq.reshape(1, o))


def kernel(inputs, table, Wq, bq):
    emb = _sc_gather(inputs, table)
    return _tc_project(emb, jnp.asarray(_PE), Wq, bq)


# speedup vs baseline: 3.6985x; 1.2330x over previous
import functools

import numpy as np

import jax
import jax.numpy as jnp
from jax import lax
from jax.experimental import pallas as pl
from jax.experimental.pallas import tpu as pltpu
from jax.experimental.pallas import tpu_sc as plsc


def _pos_encoding(seq_len, emb_size):
    seq = np.arange(1, seq_len + 1, dtype=np.float64).reshape(-1, 1)
    pos = np.arange(1, emb_size + 1, dtype=np.float64)
    pe = seq / np.power(10000.0, 2.0 * pos / emb_size)
    pe[:, ::2] = np.sin(pe[:, ::2])
    pe[:, 1::2] = np.cos(pe[:, 1::2])
    return pe.astype(np.float32)


_PE = _pos_encoding(16384, 128)


def _sc_gather(idx, table):
    info = plsc.get_sparse_core_info()
    nw = info.num_cores * info.num_subcores
    b = idx.shape[0]
    d = table.shape[1]
    b_per_w = b // nw
    ch = 128
    n_ch = b_per_w // ch

    mesh = plsc.VectorSubcoreMesh(core_axis_name="c", subcore_axis_name="s")

    @functools.partial(
        pl.kernel,
        mesh=mesh,
        out_type=jax.ShapeDtypeStruct((b, d), jnp.float32),
        scratch_types=[
            pltpu.VMEM((b_per_w,), jnp.int32),
            pltpu.VMEM((b_per_w, d), jnp.float32),
            pltpu.SemaphoreType.DMA,
        ],
    )
    def gather_kernel(idx_hbm, table_hbm, out_hbm, idx_v, rows_v, sem):
        wid = lax.axis_index("s") * info.num_cores + lax.axis_index("c")
        base = wid * b_per_w
        pltpu.sync_copy(idx_hbm.at[pl.ds(base, b_per_w)], idx_v)
        copies = [
            pltpu.async_copy(
                table_hbm.at[idx_v.at[pl.ds(j * ch, ch)]],
                rows_v.at[pl.ds(j * ch, ch)],
                sem,
            )
            for j in range(n_ch)
        ]
        for c in copies:
            c.wait()
        pltpu.sync_copy(rows_v, out_hbm.at[pl.ds(base, b_per_w)])

    return gather_kernel(idx, table)


def _tc_project(emb, pe, wq, bq):
    l, e = emb.shape
    o = wq.shape[0]
    br = 1024

    def body(emb_ref, pe_ref, wq_ref, bq_ref, out_ref):
        x = emb_ref[...] + pe_ref[...]
        out_ref[...] = (
            lax.dot_general(
                wq_ref[...], x, (((1,), (1,)), ((), ())),
                preferred_element_type=jnp.float32,
            )
            + bq_ref[...]
        )

    out_t = pl.pallas_call(
        body,
        grid=(l // br,),
        in_specs=[
            pl.BlockSpec((br, e), lambda i: (i, 0)),
            pl.BlockSpec((br, e), lambda i: (i, 0)),
            pl.BlockSpec((o, e), lambda i: (0, 0)),
            pl.BlockSpec((o, 1), lambda i: (0, 0)),
        ],
        out_specs=pl.BlockSpec((o, br), lambda i: (0, i)),
        out_shape=jax.ShapeDtypeStruct((o, l), jnp.float32),
    )(emb, pe, wq, bq.reshape(o, 1))
    return out_t.T


def kernel(inputs, table, Wq, bq):
    emb = _sc_gather(inputs, table)
    return _tc_project(emb, jnp.asarray(_PE), Wq, bq)


# speedup vs baseline: 3.7790x; 1.0218x over previous
import functools

import numpy as np

import jax
import jax.numpy as jnp
from jax import lax
from jax.experimental import pallas as pl
from jax.experimental.pallas import tpu as pltpu
from jax.experimental.pallas import tpu_sc as plsc


_BR = 1024


def _pos_tables(seq_len, emb_size, br):
    pos = np.arange(1, emb_size + 1, dtype=np.float64)
    w = 1.0 / np.power(10000.0, 2.0 * pos / emb_size)
    t = np.arange(1, br + 1, dtype=np.float64).reshape(-1, 1)
    a0 = t * w
    even = (np.arange(emb_size) % 2) == 0
    p0 = np.where(even, np.sin(a0), np.cos(a0)).astype(np.float32)
    q0 = np.where(even, np.cos(a0), -np.sin(a0)).astype(np.float32)
    off = (br * np.arange(seq_len // br, dtype=np.float64).reshape(-1, 1)) * w
    cb = np.cos(off).astype(np.float32).reshape(-1, 1, emb_size)
    sb = np.sin(off).astype(np.float32).reshape(-1, 1, emb_size)
    return p0, q0, cb, sb


_P0, _Q0, _CB, _SB = _pos_tables(16384, 128, _BR)


def _sc_gather(idx, table):
    info = plsc.get_sparse_core_info()
    nw = info.num_cores * info.num_subcores
    b = idx.shape[0]
    d = table.shape[1]
    b_per_w = b // nw
    ch = 128
    n_ch = b_per_w // ch

    mesh = plsc.VectorSubcoreMesh(core_axis_name="c", subcore_axis_name="s")

    @functools.partial(
        pl.kernel,
        mesh=mesh,
        out_type=jax.ShapeDtypeStruct((b, d), jnp.float32),
        scratch_types=[
            pltpu.VMEM((b_per_w,), jnp.int32),
            pltpu.VMEM((b_per_w, d), jnp.float32),
            pltpu.SemaphoreType.DMA,
        ],
    )
    def gather_kernel(idx_hbm, table_hbm, out_hbm, idx_v, rows_v, sem):
        wid = lax.axis_index("s") * info.num_cores + lax.axis_index("c")
        base = wid * b_per_w
        pltpu.sync_copy(idx_hbm.at[pl.ds(base, b_per_w)], idx_v)
        copies = [
            pltpu.async_copy(
                table_hbm.at[idx_v.at[pl.ds(j * ch, ch)]],
                rows_v.at[pl.ds(j * ch, ch)],
                sem,
            )
            for j in range(n_ch)
        ]
        for c in copies:
            c.wait()
        pltpu.sync_copy(rows_v, out_hbm.at[pl.ds(base, b_per_w)])

    return gather_kernel(idx, table)


def _tc_project(emb, wq, bq):
    l, e = emb.shape
    o = wq.shape[0]
    br = _BR

    def body(emb_ref, p0_ref, q0_ref, cb_ref, sb_ref, wq_ref, bq_ref,
             out_ref):
        cb = cb_ref[...].reshape(1, cb_ref.shape[-1])
        sb = sb_ref[...].reshape(1, sb_ref.shape[-1])
        pe = p0_ref[...] * cb + q0_ref[...] * sb
        x = emb_ref[...] + pe
        out_ref[...] = (
            lax.dot_general(
                wq_ref[...], x, (((1,), (1,)), ((), ())),
                preferred_element_type=jnp.float32,
            )
            + bq_ref[...]
        )

    out_t = pl.pallas_call(
        body,
        grid=(l // br,),
        in_specs=[
            pl.BlockSpec((br, e), lambda i: (i, 0)),
            pl.BlockSpec((br, e), lambda i: (0, 0)),
            pl.BlockSpec((br, e), lambda i: (0, 0)),
            pl.BlockSpec((1, 1, e), lambda i: (i, 0, 0)),
            pl.BlockSpec((1, 1, e), lambda i: (i, 0, 0)),
            pl.BlockSpec((o, e), lambda i: (0, 0)),
            pl.BlockSpec((o, 1), lambda i: (0, 0)),
        ],
        out_specs=pl.BlockSpec((o, br), lambda i: (0, i)),
        out_shape=jax.ShapeDtypeStruct((o, l), jnp.float32),
    )(emb, jnp.asarray(_P0), jnp.asarray(_Q0), jnp.asarray(_CB),
      jnp.asarray(_SB), wq, bq.reshape(o, 1))
    return out_t.T


def kernel(inputs, table, Wq, bq):
    emb = _sc_gather(inputs, table)
    return _tc_project(emb, Wq, bq)
